# Initial kernel scaffold; baseline (speedup 1.0000x reference)
import jax
import jax.numpy as jnp
from jax.experimental import pallas as pl


def kernel(audio_codes, table):
    raise NotImplementedError("write your pallas kernel here")


# speedup vs baseline: 6.4442x; 6.4442x over previous
import functools

import jax
import jax.numpy as jnp
from jax import lax
from jax.experimental import pallas as pl
from jax.experimental.pallas import tpu as pltpu
from jax.experimental.pallas import tpu_sc as plsc

HIDDEN = 2048
VOCAB = 1028
C = 9
B = 16384
D_PAD = 16

_INFO = plsc.get_sparse_core_info()
NC, NS = _INFO.num_cores, _INFO.num_subcores
NW = NC * NS
BPW = B // NW
CHUNK = 128
NCHUNK = BPW // CHUNK

_MESH = plsc.VectorSubcoreMesh(core_axis_name="c", subcore_axis_name="s")


@functools.partial(
    pl.kernel,
    out_type=jax.ShapeDtypeStruct((B, D_PAD), jnp.float32),
    mesh=_MESH,
    scratch_types=[
        pltpu.VMEM((C, NCHUNK, CHUNK), jnp.int32),
        pltpu.VMEM((BPW, D_PAD), jnp.float32),
        pltpu.SemaphoreType.DMA,
    ],
    compiler_params=pltpu.CompilerParams(use_tc_tiling_on_sc=False),
)
def _embed_sum(tokens_hbm, table_hbm, out_hbm, idx_v, acc_v, sem):
    wid = lax.axis_index("s") * NC + lax.axis_index("c")
    for c in range(C):
        pltpu.sync_copy(tokens_hbm.at[c, wid], idx_v.at[c])
    first = [
        pltpu.async_copy(
            table_hbm.at[idx_v.at[0, j]],
            acc_v.at[pl.ds(j * CHUNK, CHUNK)],
            sem,
        )
        for j in range(NCHUNK)
    ]
    for cp in first:
        cp.wait()
    rest = [
        pltpu.async_copy(
            table_hbm.at[idx_v.at[c, j]],
            acc_v.at[pl.ds(j * CHUNK, CHUNK)],
            sem,
            add=True,
        )
        for c in range(1, C)
        for j in range(NCHUNK)
    ]
    for cp in rest:
        cp.wait()
    pltpu.sync_copy(acc_v, out_hbm.at[pl.ds(wid * BPW, BPW)])


def kernel(audio_codes, table):
    codes = audio_codes.reshape(B, C)
    compact = table[: C * HIDDEN].reshape(C, HIDDEN, C)[:, :VOCAB, :]
    compact = jnp.pad(compact, ((0, 0), (0, 0), (0, D_PAD - C)))
    compact = compact.reshape(C * VOCAB, D_PAD)
    tokens = codes + jnp.arange(C, dtype=codes.dtype) * VOCAB
    tokens = tokens.T.reshape(C, NW, NCHUNK, CHUNK)
    out = _embed_sum(tokens, compact)
    return out[:, :C].reshape(B, 1, C)
